# Initial kernel scaffold; baseline (speedup 1.0000x reference)
import jax
import jax.numpy as jnp
from jax.experimental import pallas as pl


def kernel(x, edge_index, batch, W1, b1, g1, be1, W2, b2, g2, be2, W3, a_src, a_dst, b3, g3, be3, Wf1, bf1, Wf2, bf2):
    raise NotImplementedError("write your pallas kernel here")


# speedup vs baseline: 1.0071x; 1.0071x over previous
import jax
import jax.numpy as jnp
from jax.experimental import pallas as pl
from jax.experimental.pallas import tpu as pltpu

N = 10000
G = 64
EPS = 1e-5


def _final_body(h_ref, batch_ref, wf1_ref, bf1_ref, wf2_ref, bf2_ref, o_ref):
    h = h_ref[...]
    batch = batch_ref[...]
    gids = jax.lax.broadcasted_iota(jnp.int32, (G, N), 0)
    P = (batch == gids).astype(jnp.float32)
    sums = jnp.dot(P, h, preferred_element_type=jnp.float32)
    counts = jnp.sum(P, axis=1, keepdims=True)
    pooled = sums / jnp.maximum(counts, 1.0)
    z = jnp.maximum(jnp.dot(pooled, wf1_ref[...], preferred_element_type=jnp.float32)
                    + bf1_ref[...], 0.0)
    z = jnp.dot(z, wf2_ref[...], preferred_element_type=jnp.float32) + bf2_ref[...]
    m = jnp.max(z, axis=1, keepdims=True)
    lse = jnp.log(jnp.sum(jnp.exp(z - m), axis=1, keepdims=True)) + m
    o_ref[...] = z - lse


def _final_stage(h, batch, Wf1, bf1, Wf2, bf2):
    return pl.pallas_call(
        _final_body,
        out_shape=jax.ShapeDtypeStruct((G, 31), jnp.float32),
    )(h, batch.reshape(1, N), Wf1, bf1.reshape(1, -1), Wf2, bf2.reshape(1, -1))


def _gcn(x, src, dst, W, b, n):
    h = x @ W
    deg = jnp.zeros((n,), jnp.float32).at[dst].add(1.0)
    dinv = jnp.where(deg > 0, 1.0 / jnp.sqrt(deg), 0.0)
    norm = dinv[src] * dinv[dst]
    msg = h[src] * norm[:, None]
    out = jnp.zeros((n, h.shape[1]), jnp.float32).at[dst].add(msg)
    return out + b


def _gat(x, src, dst, W, a_src, a_dst, b, n):
    h = x @ W
    asrc = h @ a_src
    adst = h @ a_dst
    e = asrc[src] + adst[dst]
    e = jnp.where(e > 0, e, 0.2 * e)
    m = jax.ops.segment_max(e, dst, num_segments=n)
    m = jnp.where(jnp.isfinite(m), m, 0.0)
    ex = jnp.exp(e - m[dst])
    denom = jax.ops.segment_sum(ex, dst, num_segments=n)
    alpha = ex / jnp.maximum(denom[dst], 1e-16)
    out = jax.ops.segment_sum(h[src] * alpha[:, None], dst, num_segments=n)
    return out + b


def _bn(x, g, b):
    mu = jnp.mean(x, axis=0)
    var = jnp.var(x, axis=0)
    return g * (x - mu) / jnp.sqrt(var + EPS) + b


def kernel(x, edge_index, batch, W1, b1, g1, be1, W2, b2, g2, be2, W3, a_src, a_dst, b3, g3, be3, Wf1, bf1, Wf2, bf2):
    n = x.shape[0]
    loops = jnp.arange(n, dtype=edge_index.dtype)
    src = jnp.concatenate([edge_index[0], loops])
    dst = jnp.concatenate([edge_index[1], loops])
    h = jax.nn.relu(_gcn(x, src, dst, W1, b1, n))
    h = _bn(h, g1, be1)
    h = jax.nn.relu(_gcn(h, src, dst, W2, b2, n))
    h = _bn(h, g2, be2)
    h = jax.nn.elu(_gat(h, src, dst, W3, a_src, a_dst, b3, n))
    h = _bn(h, g3, be3)
    return _final_stage(h, batch, Wf1, bf1, Wf2, bf2)


# speedup vs baseline: 8.4119x; 8.3529x over previous
import functools

import jax
import jax.numpy as jnp
from jax import lax
from jax.experimental import pallas as pl
from jax.experimental.pallas import tpu as pltpu
from jax.experimental.pallas import tpu_sc as plsc

N = 10000
G = 64
EPS = 1e-5
NC, NS, L = 2, 16, 16
NW = NC * NS
CHUNK = 128
E = 320000
ME = E + N
CPW = -(-ME // (NW * CHUNK))
MEP = NW * CPW * CHUNK
NPAD = N + 112
RPT = NPAD // NS
FC = 128

_TC_PARAMS = pltpu.CompilerParams(vmem_limit_bytes=160 * 1024 * 1024)
_SC_PARAMS = pltpu.CompilerParams(needs_layout_passes=False)


def _sc_mesh():
    return plsc.VectorSubcoreMesh(core_axis_name="c", subcore_axis_name="s")


def _deg_body(dstp_hbm, out_hbm, hist, idxb):
    cid = lax.axis_index("c")
    sid = lax.axis_index("s")
    wid = sid * NC + cid
    zv = jnp.zeros((L,), jnp.float32)

    def zb(i, c):
        hist[pl.ds(i * L, L)] = zv
        return c
    lax.fori_loop(0, NPAD // L, zb, 0)

    onev = jnp.ones((L,), jnp.float32)

    def cb(j, c):
        off = (wid * CPW + j) * CHUNK
        pltpu.sync_copy(dstp_hbm.at[pl.ds(off, CHUNK)], idxb)
        for v in range(CHUNK // L):
            d16 = idxb[pl.ds(v * L, L)]
            plsc.addupdate_scatter(hist, [d16], onev)
        return c
    lax.fori_loop(0, CPW, cb, 0)
    pltpu.sync_copy(hist, out_hbm.at[pl.ds(wid * NPAD, NPAD)])


def _deg_call(dstp):
    return pl.kernel(
        _deg_body,
        out_type=jax.ShapeDtypeStruct((NW * NPAD,), jnp.float32),
        mesh=_sc_mesh(),
        compiler_params=_SC_PARAMS,
        scratch_types=[
            pltpu.VMEM((NPAD,), jnp.float32),
            pltpu.VMEM((CHUNK,), jnp.int32),
        ],
    )(dstp)


def _msg_body(weighted, h_hbm, srcp_hbm, dstp_hbm, w_hbm, zrows_hbm, out_hbm,
              acc, srcb, dstb, wb, rows, sem):
    cid = lax.axis_index("c")
    sid = lax.axis_index("s")
    wid = sid * NC + cid

    r0 = sid * RPT
    pltpu.sync_copy(zrows_hbm.at[pl.ds(r0, RPT)], acc.at[pl.ds(r0, RPT)])
    plsc.subcore_barrier()

    def cb(j, c):
        off = (wid * CPW + j) * CHUNK
        pltpu.sync_copy(srcp_hbm.at[pl.ds(off, CHUNK)], srcb)
        pltpu.async_copy(h_hbm.at[srcb], rows, sem).wait()
        pltpu.sync_copy(dstp_hbm.at[pl.ds(off, CHUNK)], dstb)
        if weighted:
            pltpu.sync_copy(w_hbm.at[pl.ds(off, CHUNK)], wb)

            def sb(e, c2):
                wsp = plsc.load_gather(wb, [jnp.zeros((L,), jnp.int32) + e])
                for v in range(FC // L):
                    rows[e, pl.ds(v * L, L)] = rows[e, pl.ds(v * L, L)] * wsp
                return c2
            lax.fori_loop(0, CHUNK, sb, 0)
        pltpu.sync_copy(rows, acc.at[dstb], add=True)
        return c
    lax.fori_loop(0, CPW, cb, 0)
    plsc.subcore_barrier()

    pltpu.sync_copy(acc.at[pl.ds(r0, RPT)], out_hbm.at[cid, pl.ds(r0, RPT)])


def _make_msg(weighted):
    body = functools.partial(_msg_body, weighted)
    if not weighted:
        def body2(h, s, d, z, o, *rest):
            return functools.partial(_msg_body, False)(h, s, d, None, z, o, *rest)
        body = body2
    return pl.kernel(
        body,
        out_type=jax.ShapeDtypeStruct((NC, NPAD, FC), jnp.float32),
        mesh=_sc_mesh(),
        compiler_params=_SC_PARAMS,
        scratch_types=[
            pltpu.VMEM_SHARED((NPAD, FC), jnp.float32),
            pltpu.VMEM((CHUNK,), jnp.int32),
            pltpu.VMEM((CHUNK,), jnp.int32),
            pltpu.VMEM((CHUNK,), jnp.float32),
            pltpu.VMEM((CHUNK, FC), jnp.float32),
            pltpu.SemaphoreType.DMA,
        ],
    )


def _msg_call(h, srcp, dstp, zrows, w=None):
    if w is None:
        return _make_msg(False)(h, srcp, dstp, zrows)
    return _make_msg(True)(h, srcp, dstp, w, zrows)


def _gat_body(srcp_hbm, dstp_hbm, asrc_hbm, adst_hbm, ex_hbm, den_hbm,
              asrcT, adstT, hist, srcb, dstb, exb):
    cid = lax.axis_index("c")
    sid = lax.axis_index("s")
    wid = sid * NC + cid
    pltpu.sync_copy(asrc_hbm, asrcT)
    pltpu.sync_copy(adst_hbm, adstT)
    zv = jnp.zeros((L,), jnp.float32)

    def zb(i, c):
        hist[pl.ds(i * L, L)] = zv
        return c
    lax.fori_loop(0, NPAD // L, zb, 0)

    def cb(j, c):
        off = (wid * CPW + j) * CHUNK
        pltpu.sync_copy(srcp_hbm.at[pl.ds(off, CHUNK)], srcb)
        pltpu.sync_copy(dstp_hbm.at[pl.ds(off, CHUNK)], dstb)
        for v in range(CHUNK // L):
            s16 = srcb[pl.ds(v * L, L)]
            d16 = dstb[pl.ds(v * L, L)]
            e = plsc.load_gather(asrcT, [s16]) + plsc.load_gather(adstT, [d16])
            e = jnp.where(e > 0, e, 0.2 * e)
            ex = jnp.exp(e)
            plsc.addupdate_scatter(hist, [d16], ex)
            exb[pl.ds(v * L, L)] = ex
        pltpu.sync_copy(exb, ex_hbm.at[pl.ds(off, CHUNK)])
        return c
    lax.fori_loop(0, CPW, cb, 0)
    pltpu.sync_copy(hist, den_hbm.at[pl.ds(wid * NPAD, NPAD)])


def _gat_call(srcp, dstp, asrcp, adstp):
    return pl.kernel(
        _gat_body,
        out_type=(jax.ShapeDtypeStruct((MEP,), jnp.float32),
                  jax.ShapeDtypeStruct((NW * NPAD,), jnp.float32)),
        mesh=_sc_mesh(),
        compiler_params=_SC_PARAMS,
        scratch_types=[
            pltpu.VMEM((NPAD,), jnp.float32),
            pltpu.VMEM((NPAD,), jnp.float32),
            pltpu.VMEM((NPAD,), jnp.float32),
            pltpu.VMEM((CHUNK,), jnp.int32),
            pltpu.VMEM((CHUNK,), jnp.int32),
            pltpu.VMEM((CHUNK,), jnp.float32),
        ],
    )(srcp, dstp, asrcp, adstp)


def _tc0_body(x_ref, w1_ref, degT_ref, hs1_ref, dinv_ref):
    deg = jnp.sum(degT_ref[...], axis=1, keepdims=True)
    dinv = jnp.where(deg > 0, lax.rsqrt(deg), 0.0)
    h = jnp.dot(x_ref[...], w1_ref[...], preferred_element_type=jnp.float32)
    hs1_ref[...] = h * dinv[:N]
    dinv_ref[...] = dinv


def _tc0(x, W1, degT):
    return pl.pallas_call(
        _tc0_body,
        out_shape=(jax.ShapeDtypeStruct((N, 128), jnp.float32),
                   jax.ShapeDtypeStruct((NPAD, 1), jnp.float32)),
        compiler_params=_TC_PARAMS,
    )(x, W1, degT)


def _bn(h, g, be):
    mu = jnp.mean(h, axis=0, keepdims=True)
    var = jnp.mean((h - mu) ** 2, axis=0, keepdims=True)
    return g * (h - mu) * lax.rsqrt(var + EPS) + be


def _tcl1_body(a_ref, dinv_ref, b1_ref, g1_ref, be1_ref, w2_ref, o0_ref, o1_ref):
    a3 = a_ref[...]
    A = (a3[0] + a3[1])[:N]
    dinv = dinv_ref[...][:N]
    h = jnp.maximum(A * dinv + b1_ref[...], 0.0)
    hn = _bn(h, g1_ref[...], be1_ref[...])
    h2 = jnp.dot(hn, w2_ref[...], preferred_element_type=jnp.float32) * dinv
    o0_ref[...] = h2[:, :FC]
    o1_ref[...] = h2[:, FC:]


def _tcl1(a1, dinv, b1, g1, be1, W2):
    return pl.pallas_call(
        _tcl1_body,
        out_shape=(jax.ShapeDtypeStruct((N, FC), jnp.float32),
                   jax.ShapeDtypeStruct((N, FC), jnp.float32)),
        compiler_params=_TC_PARAMS,
    )(a1, dinv, b1.reshape(1, -1), g1.reshape(1, -1), be1.reshape(1, -1), W2)


def _tcbn2_body(a0_ref, a1_ref, dinv_ref, b2_ref, g2_ref, be2_ref,
                o0_ref, o1_ref):
    dinv = dinv_ref[...][:N]
    for a_ref, o_ref, c in ((a0_ref, o0_ref, 0), (a1_ref, o1_ref, 1)):
        p = a_ref[...]
        A = (p[0] + p[1])[:N]
        sl = slice(c * FC, (c + 1) * FC)
        h = jnp.maximum(A * dinv + b2_ref[...][:, sl], 0.0)
        o_ref[...] = _bn(h, g2_ref[...][:, sl], be2_ref[...][:, sl])


def _tcbn2(a2c0, a2c1, dinv, b2, g2, be2):
    return pl.pallas_call(
        _tcbn2_body,
        out_shape=(jax.ShapeDtypeStruct((N, FC), jnp.float32),
                   jax.ShapeDtypeStruct((N, FC), jnp.float32)),
        compiler_params=_TC_PARAMS,
    )(a2c0, a2c1, dinv, b2.reshape(1, -1), g2.reshape(1, -1), be2.reshape(1, -1))


def _tcmm2_body(h0_ref, h1_ref, w3_ref, avs_ref, avd_ref,
                o0_ref, o1_ref, o2_ref, o3_ref, asrc_ref, adst_ref):
    hn0 = h0_ref[...]
    hn1 = h1_ref[...]
    w3 = w3_ref[...]
    for c, o_ref in enumerate((o0_ref, o1_ref, o2_ref, o3_ref)):
        sl = slice(c * FC, (c + 1) * FC)
        o_ref[...] = (jnp.dot(hn0, w3[:FC, sl], preferred_element_type=jnp.float32)
                      + jnp.dot(hn1, w3[FC:, sl], preferred_element_type=jnp.float32))
    was = jnp.dot(w3, avs_ref[...], preferred_element_type=jnp.float32)
    wad = jnp.dot(w3, avd_ref[...], preferred_element_type=jnp.float32)
    asrc = (jnp.dot(hn0, was[:FC], preferred_element_type=jnp.float32)
            + jnp.dot(hn1, was[FC:], preferred_element_type=jnp.float32))
    adst = (jnp.dot(hn0, wad[:FC], preferred_element_type=jnp.float32)
            + jnp.dot(hn1, wad[FC:], preferred_element_type=jnp.float32))
    zpad = jnp.zeros((NPAD - N, 1), jnp.float32)
    asrc_ref[...] = jnp.concatenate([asrc, zpad], axis=0)
    adst_ref[...] = jnp.concatenate([adst, zpad], axis=0)


def _tcmm2(hn2c0, hn2c1, W3, a_src, a_dst):
    return pl.pallas_call(
        _tcmm2_body,
        out_shape=(jax.ShapeDtypeStruct((N, FC), jnp.float32),
                   jax.ShapeDtypeStruct((N, FC), jnp.float32),
                   jax.ShapeDtypeStruct((N, FC), jnp.float32),
                   jax.ShapeDtypeStruct((N, FC), jnp.float32),
                   jax.ShapeDtypeStruct((NPAD, 1), jnp.float32),
                   jax.ShapeDtypeStruct((NPAD, 1), jnp.float32)),
        compiler_params=_TC_PARAMS,
    )(hn2c0, hn2c1, W3, a_src.reshape(-1, 1), a_dst.reshape(-1, 1))


def _tcpost_body(a_ref, denT_ref, b3c_ref, o_ref):
    p = a_ref[...]
    A = (p[0] + p[1])[:N]
    denom = jnp.sum(denT_ref[...], axis=1, keepdims=True)[:N]
    h = A / jnp.maximum(denom, 1e-16) + b3c_ref[...]
    o_ref[...] = jnp.where(h > 0, h, jnp.exp(jnp.minimum(h, 0.0)) - 1.0)


def _tcpost(a3ci, denT, b3c):
    return pl.pallas_call(
        _tcpost_body,
        out_shape=jax.ShapeDtypeStruct((N, FC), jnp.float32),
        compiler_params=_TC_PARAMS,
    )(a3ci, denT, b3c)


def _tcfinal_body(h0_ref, h1_ref, h2_ref, h3_ref, g3_ref, be3_ref, batch_ref,
                  wf1_ref, bf1_ref, wf2_ref, bf2_ref, o_ref):
    batch = batch_ref[...][:N]
    gids = lax.broadcasted_iota(jnp.int32, (N, G), 1)
    PT = (batch == gids).astype(jnp.float32)
    dims = (((0,), (0,)), ((), ()))
    counts = lax.dot_general(PT, jnp.ones((N, 1), jnp.float32), dims,
                             preferred_element_type=jnp.float32)
    cdiv = 1.0 / jnp.maximum(counts, 1.0)
    z1 = jnp.zeros((G, 256), jnp.float32)
    for c, h_ref in enumerate((h0_ref, h1_ref, h2_ref, h3_ref)):
        sl = slice(c * FC, (c + 1) * FC)
        hn = _bn(h_ref[...], g3_ref[...][:, sl], be3_ref[...][:, sl])
        pooled = lax.dot_general(PT, hn, dims,
                                 preferred_element_type=jnp.float32) * cdiv
        z1 = z1 + jnp.dot(pooled, wf1_ref[...][sl, :],
                          preferred_element_type=jnp.float32)
    z = jnp.maximum(z1 + bf1_ref[...], 0.0)
    z = jnp.dot(z, wf2_ref[...], preferred_element_type=jnp.float32) + bf2_ref[...]
    m = jnp.max(z, axis=1, keepdims=True)
    lse = jnp.log(jnp.sum(jnp.exp(z - m), axis=1, keepdims=True)) + m
    o_ref[...] = z - lse


def _tcfinal(h3e, g3, be3, batchp, Wf1, bf1, Wf2, bf2):
    return pl.pallas_call(
        _tcfinal_body,
        out_shape=jax.ShapeDtypeStruct((G, 31), jnp.float32),
        compiler_params=_TC_PARAMS,
    )(h3e[0], h3e[1], h3e[2], h3e[3], g3.reshape(1, -1), be3.reshape(1, -1),
      batchp, Wf1, bf1.reshape(1, -1), Wf2, bf2.reshape(1, -1))


def kernel(x, edge_index, batch, W1, b1, g1, be1, W2, b2, g2, be2, W3, a_src,
           a_dst, b3, g3, be3, Wf1, bf1, Wf2, bf2):
    loops = jnp.arange(N, dtype=jnp.int32)
    srcp = jnp.concatenate([edge_index[0], loops,
                            jnp.zeros((MEP - ME,), jnp.int32)])
    dstp = jnp.concatenate([edge_index[1], loops,
                            jnp.full((MEP - ME,), N, jnp.int32)])
    zrows = jnp.zeros((NPAD, FC), jnp.float32)

    deg32 = _deg_call(dstp).reshape(NW, NPAD)
    hs1, dinv = _tc0(x, W1, deg32.T)

    a1 = _msg_call(hs1, srcp, dstp, zrows)
    h2c0, h2c1 = _tcl1(a1, dinv, b1, g1, be1, W2)

    a2c0 = _msg_call(h2c0, srcp, dstp, zrows)
    a2c1 = _msg_call(h2c1, srcp, dstp, zrows)
    hn2c0, hn2c1 = _tcbn2(a2c0, a2c1, dinv, b2, g2, be2)
    h3c0, h3c1, h3c2, h3c3, asrcp, adstp = _tcmm2(hn2c0, hn2c1, W3, a_src, a_dst)

    exw, den32 = _gat_call(srcp, dstp, asrcp.reshape(NPAD), adstp.reshape(NPAD))
    denT = den32.reshape(NW, NPAD).T
    a3c = [_msg_call(h, srcp, dstp, zrows, w=exw)
           for h in (h3c0, h3c1, h3c2, h3c3)]

    b3r = b3.reshape(1, -1)
    h3e = [_tcpost(a3c[c], denT, b3r[:, c * FC:(c + 1) * FC]) for c in range(4)]
    batchp = jnp.concatenate([batch.astype(jnp.int32),
                              jnp.full((NPAD - N,), G, jnp.int32)]).reshape(NPAD, 1)
    return _tcfinal(h3e, g3, be3, batchp, Wf1, bf1, Wf2, bf2)
